# baseline (speedup 1.0000x reference)
import functools

import jax
import jax.numpy as jnp
from jax import lax
from jax.experimental import pallas as pl
from jax.experimental.pallas import tpu as pltpu
from jax.experimental.pallas import tpu_sc as plsc

_R = 1024
_C = 100000
_K = 200
_NW = 32
_L = 16
_NA = _K // 8
_NC = _R // 128
_AMAX = 7
_A0 = (0, 7, 13, 19)


@functools.partial(
    pl.kernel,
    mesh=plsc.VectorSubcoreMesh(core_axis_name="c", subcore_axis_name="s"),
    out_type=jax.ShapeDtypeStruct((_NA, _NC, 8, 128), jnp.float32),
    scratch_types=[
        pltpu.VMEM((_AMAX * 8 * 128,), jnp.int32),
        pltpu.VMEM((_AMAX * 8, 128), jnp.int32),
        pltpu.VMEM((_AMAX * 8, 128), jnp.float32),
        pltpu.SemaphoreType.DMA,
        pltpu.SemaphoreType.DMA,
    ],
)
def _sc_gather(tbl_hbm, idx_hbm, out_hbm, idx_v, pidx_v, out_v, sem, sem_wb):
    cid = lax.axis_index("c")
    sid = lax.axis_index("s")
    wid = sid * 2 + cid
    c0 = wid & 7
    t = wid >> 3
    a0 = jnp.where(t == 0, 0, t * 6 + 1)
    na = jnp.where(t == 0, 7, 6)

    def stage(k, carry):
        pltpu.async_copy(
            idx_hbm.at[pl.ds(((a0 + k) * _NC + c0) * 1024, 1024)],
            idx_v.at[pl.ds(k * 1024, 1024)], sem)
        return carry

    lax.fori_loop(0, na, stage, 0)

    def conv_fire(k, carry):
        pltpu.make_async_copy(
            idx_hbm.at[pl.ds(((a0 + k) * _NC + c0) * 1024, 1024)],
            idx_v.at[pl.ds(k * 1024, 1024)], sem).wait()
        for b in range(8):
            jl = k * 8 + b
            for v in range(128 // _L):
                sbase = (c0 << 10) + v * _L + lax.iota(jnp.int32, _L)
                q = idx_v[pl.ds(jl * 128 + v * _L, _L)]
                pidx_v[jl, pl.ds(v * _L, _L)] = (
                    ((q >> 3) << 13) + ((q & 7) << 7) + sbase)
            pltpu.async_copy(tbl_hbm.at[pidx_v.at[jl]], out_v.at[jl], sem)
        return carry

    lax.fori_loop(0, na, conv_fire, 0)

    def drain_wb(k, carry):
        for b in range(8):
            jl = k * 8 + b
            pltpu.make_async_copy(
                tbl_hbm.at[pidx_v.at[jl]], out_v.at[jl], sem).wait()
        pltpu.async_copy(out_v.at[pl.ds(k * 8, 8)], out_hbm.at[a0 + k, c0],
                         sem_wb)
        return carry

    lax.fori_loop(0, na, drain_wb, 0)

    def wb_wait(k, carry):
        pltpu.make_async_copy(out_v.at[pl.ds(k * 8, 8)],
                              out_hbm.at[a0 + k, c0], sem_wb).wait()
        return carry

    lax.fori_loop(0, na, wb_wait, 0)


def kernel(input_tensor, indices):
    tbl = (input_tensor.T.reshape(_C // 8, 8, _R // 128, 128)
           .transpose(0, 2, 1, 3).reshape(-1))
    idx = (indices.T.reshape(_NA, 8, _NC, 128)
           .transpose(0, 2, 1, 3).reshape(-1))
    out = _sc_gather(tbl, idx)
    return out.transpose(0, 2, 1, 3).reshape(_K, _R).T


# speedup vs baseline: 1.0310x; 1.0310x over previous
import functools

import jax
import jax.numpy as jnp
from jax import lax
from jax.experimental import pallas as pl
from jax.experimental.pallas import tpu as pltpu
from jax.experimental.pallas import tpu_sc as plsc

_R = 1024
_C = 100000
_K = 200
_NW = 32
_L = 16
_NA = _K // 8
_NC = _R // 128
_AMAX = 7
_A0 = (0, 7, 13, 19)


@functools.partial(
    pl.kernel,
    mesh=plsc.VectorSubcoreMesh(core_axis_name="c", subcore_axis_name="s"),
    out_type=jax.ShapeDtypeStruct((_NA, _NC, 8, 128), jnp.float32),
    scratch_types=[
        pltpu.VMEM((_AMAX * 8 * 128,), jnp.int32),
        pltpu.VMEM((_AMAX * 8, 128), jnp.int32),
        pltpu.VMEM((_AMAX * 8, 128), jnp.float32),
        pltpu.SemaphoreType.DMA,
        pltpu.SemaphoreType.DMA,
    ],
)
def _sc_gather(tbl_hbm, idx_hbm, out_hbm, idx_v, pidx_v, out_v, sem, sem_wb):
    cid = lax.axis_index("c")
    sid = lax.axis_index("s")
    wid = sid * 2 + cid
    c0 = wid & 7
    t = wid >> 3
    a0 = jnp.where(t == 0, 0, t * 6 + 1)
    na = jnp.where(t == 0, 7, 6)

    def stage(k, carry):
        pltpu.async_copy(
            idx_hbm.at[pl.ds(((a0 + k) * _NC + c0) * 1024, 1024)],
            idx_v.at[pl.ds(k * 1024, 1024)], sem)
        return carry

    lax.fori_loop(0, na, stage, 0)

    def stage_wait(k, carry):
        pltpu.make_async_copy(
            idx_hbm.at[pl.ds(((a0 + k) * _NC + c0) * 1024, 1024)],
            idx_v.at[pl.ds(k * 1024, 1024)], sem).wait()
        return carry

    lax.fori_loop(0, na, stage_wait, 0)

    def conv_fire(k, carry):
        for b in range(8):
            jl = k * 8 + b
            for v in range(128 // _L):
                sbase = (c0 << 10) + v * _L + lax.iota(jnp.int32, _L)
                q = idx_v[pl.ds(jl * 128 + v * _L, _L)]
                pidx_v[jl, pl.ds(v * _L, _L)] = (
                    ((q >> 3) << 13) + ((q & 7) << 7) + sbase)
            pltpu.async_copy(tbl_hbm.at[pidx_v.at[jl]], out_v.at[jl], sem)
        return carry

    lax.fori_loop(0, na, conv_fire, 0)

    def drain_wb(k, carry):
        for b in range(8):
            jl = k * 8 + b
            pltpu.make_async_copy(
                tbl_hbm.at[pidx_v.at[jl]], out_v.at[jl], sem).wait()
        pltpu.async_copy(out_v.at[pl.ds(k * 8, 8)], out_hbm.at[a0 + k, c0],
                         sem_wb)
        return carry

    lax.fori_loop(0, na, drain_wb, 0)

    def wb_wait(k, carry):
        pltpu.make_async_copy(out_v.at[pl.ds(k * 8, 8)],
                              out_hbm.at[a0 + k, c0], sem_wb).wait()
        return carry

    lax.fori_loop(0, na, wb_wait, 0)


def kernel(input_tensor, indices):
    tbl = (input_tensor.T.reshape(_C // 8, 8, _R // 128, 128)
           .transpose(0, 2, 1, 3).reshape(-1))
    idx = (indices.T.reshape(_NA, 8, _NC, 128)
           .transpose(0, 2, 1, 3).reshape(-1))
    out = _sc_gather(tbl, idx)
    return out.transpose(0, 2, 1, 3).reshape(_K, _R).T
